# baseline (speedup 1.0000x reference)
import jax
import jax.numpy as jnp
from jax.experimental import pallas as pl
from jax.experimental.pallas import tpu as pltpu

M1 = 4096
M2 = 4096
D = 256
DP = 384

BM = 512
BN = 2048

LOG2E = 1.4426950408889634


def _augment(x, scale_features):
    n = (-0.5 * LOG2E) * jnp.sum(x * x, axis=1, keepdims=True)
    lane = jax.lax.broadcasted_iota(jnp.int32, (x.shape[0], DP - D), 1)
    if scale_features:
        extra = jnp.where(lane == 0, n, jnp.where(lane == 1, 1.0, 0.0))
        body = x * LOG2E
    else:
        extra = jnp.where(lane == 0, 1.0, jnp.where(lane == 1, n, 0.0))
        body = x
    return jnp.concatenate([body, extra], axis=1).astype(jnp.bfloat16)


def _tile_body(x1_ref, x2_ref, k_ref, keep_ref, bt_ref, m_ref):
    i = pl.program_id(0)
    j = pl.program_id(1)

    @pl.when(i == 0)
    def _prep():
        bt_ref[pl.ds(j * BN, BN), :] = _augment(x2_ref[...], scale_features=False)
        keep_ref[...] = jnp.ones((1, BN), jnp.int32)

    at = _augment(x1_ref[...], scale_features=True)
    arg = jax.lax.dot_general(
        at, bt_ref[pl.ds(j * BN, BN), :], (((1,), (1,)), ((), ())),
        preferred_element_type=jnp.float32,
    )
    k = jnp.exp2(arg)
    k_ref[...] = k

    full_max = jnp.max(k, axis=0, keepdims=True)
    gl = j * BN + jax.lax.broadcasted_iota(jnp.int32, (1, BN), 1)
    m_ref[...] = jnp.where(gl >= (i + 1) * BM, full_max, 0.0)

    @pl.when(j == (i * BM) // BN)
    def _band():
        off = i * BM - ((i * BM) // BN) * BN
        band = k_ref[:, pl.ds(off, BM)]
        tri = (jax.lax.broadcasted_iota(jnp.int32, (BM, BM), 0)
               <= jax.lax.broadcasted_iota(jnp.int32, (BM, BM), 1))
        band_max = jnp.max(jnp.where(tri, band, 0.0), axis=0, keepdims=True)
        m_ref[:, pl.ds(off, BM)] = band_max

    keep_ref[...] &= (m_ref[...] < 1.0).astype(jnp.int32)


@jax.jit
def kernel(x1, x2):
    k_mat, keep_i32 = pl.pallas_call(
        _tile_body,
        grid=(M1 // BM, M2 // BN),
        in_specs=[
            pl.BlockSpec((BM, D), lambda i, j: (i, 0)),
            pl.BlockSpec((BN, D), lambda i, j: (j, 0)),
        ],
        out_specs=[
            pl.BlockSpec((BM, BN), lambda i, j: (i, j)),
            pl.BlockSpec((1, BN), lambda i, j: (0, j)),
        ],
        out_shape=[
            jax.ShapeDtypeStruct((M1, M2), jnp.float32),
            jax.ShapeDtypeStruct((1, M2), jnp.int32),
        ],
        scratch_shapes=[
            pltpu.VMEM((M2, DP), jnp.bfloat16),
            pltpu.VMEM((1, BN), jnp.float32),
        ],
        compiler_params=pltpu.CompilerParams(
            dimension_semantics=("arbitrary", "arbitrary"),
        ),
    )(x1, x2)
    keep_mask = keep_i32[0].astype(bool)
    return k_mat, keep_mask


# speedup vs baseline: 1.3404x; 1.3404x over previous
import jax
import jax.numpy as jnp
from jax.experimental import pallas as pl
from jax.experimental.pallas import tpu as pltpu

M1 = 4096
M2 = 4096
D = 256
DP = 384

BM = 512

LOG2E = 1.4426950408889634


def _augment(x, scale_features):
    n = (-0.5 * LOG2E) * jnp.sum(x * x, axis=1, keepdims=True)
    lane = jax.lax.broadcasted_iota(jnp.int32, (x.shape[0], DP - D), 1)
    if scale_features:
        extra = jnp.where(lane == 0, n, jnp.where(lane == 1, 1.0, 0.0))
        body = x * LOG2E
    else:
        extra = jnp.where(lane == 0, 1.0, jnp.where(lane == 1, n, 0.0))
        body = x
    return jnp.concatenate([body, extra], axis=1).astype(jnp.bfloat16)


def _tile_body(x1_ref, x2_ref, k_ref, keep_ref, bt_ref, m_ref):
    i = pl.program_id(0)

    @pl.when(i == 0)
    def _prep():
        bt_ref[...] = _augment(x2_ref[...], scale_features=False)
        keep_ref[...] = jnp.ones((1, M2), jnp.int32)

    at = _augment(x1_ref[...], scale_features=True)
    arg = jax.lax.dot_general(
        at, bt_ref[...], (((1,), (1,)), ((), ())),
        preferred_element_type=jnp.float32,
    )
    k = jnp.exp2(arg)
    k_ref[...] = k

    full_max = jnp.max(k, axis=0, keepdims=True)
    lanes = jax.lax.broadcasted_iota(jnp.int32, (1, M2), 1)
    m_ref[...] = jnp.where(lanes >= (i + 1) * BM, full_max, 0.0)
    band = k_ref[:, pl.ds(i * BM, BM)]
    tri = (jax.lax.broadcasted_iota(jnp.int32, (BM, BM), 0)
           <= jax.lax.broadcasted_iota(jnp.int32, (BM, BM), 1))
    band_max = jnp.max(jnp.where(tri, band, 0.0), axis=0, keepdims=True)
    m_ref[:, pl.ds(i * BM, BM)] = band_max
    keep_ref[...] &= (m_ref[...] < 1.0).astype(jnp.int32)


@jax.jit
def kernel(x1, x2):
    k_mat, keep_i32 = pl.pallas_call(
        _tile_body,
        grid=(M1 // BM,),
        in_specs=[
            pl.BlockSpec((BM, D), lambda i: (i, 0)),
            pl.BlockSpec((M2, D), lambda i: (0, 0)),
        ],
        out_specs=[
            pl.BlockSpec((BM, M2), lambda i: (i, 0)),
            pl.BlockSpec((1, M2), lambda i: (0, 0)),
        ],
        out_shape=[
            jax.ShapeDtypeStruct((M1, M2), jnp.float32),
            jax.ShapeDtypeStruct((1, M2), jnp.int32),
        ],
        scratch_shapes=[
            pltpu.VMEM((M2, DP), jnp.bfloat16),
            pltpu.VMEM((1, M2), jnp.float32),
        ],
        compiler_params=pltpu.CompilerParams(
            dimension_semantics=("arbitrary",),
        ),
    )(x1, x2)
    keep_mask = keep_i32[0].astype(bool)
    return k_mat, keep_mask
